# baseline (speedup 1.0000x reference)
import functools

import jax
import jax.numpy as jnp
from jax import lax
from jax.experimental import pallas as pl
from jax.experimental.pallas import tpu as pltpu
from jax.experimental.pallas import tpu_sc as plsc

_NC, _NS = 2, 16
_NW = _NC * _NS
_B = 16384
_CHUNK = 128
_NCHUNK = _B // _CHUNK
_CPW = _NCHUNK // _NW


def _sc_gather(uidx2, iidx2, tidx2, due, die, dte, wue, wie):
    mesh = plsc.VectorSubcoreMesh(core_axis_name="c", subcore_axis_name="s")

    @functools.partial(
        pl.kernel,
        compiler_params=pltpu.CompilerParams(use_tc_tiling_on_sc=False),
        out_type=[
            jax.ShapeDtypeStruct((_NCHUNK, _CHUNK, 32), jnp.float32),
            jax.ShapeDtypeStruct((_NCHUNK, _CHUNK, 32), jnp.float32),
            jax.ShapeDtypeStruct((_NCHUNK, _CHUNK, 8), jnp.float32),
            jax.ShapeDtypeStruct((_NCHUNK, _CHUNK, 1), jnp.float32),
            jax.ShapeDtypeStruct((_NCHUNK, _CHUNK, 1), jnp.float32),
        ],
        mesh=mesh,
        scratch_types=[
            pltpu.VMEM((_CPW, _CHUNK), jnp.int32),
            pltpu.VMEM((_CPW, _CHUNK), jnp.int32),
            pltpu.VMEM((_CPW, _CHUNK), jnp.int32),
            pltpu.VMEM((_CPW, _CHUNK, 32), jnp.float32),
            pltpu.VMEM((_CPW, _CHUNK, 32), jnp.float32),
            pltpu.VMEM((_CPW, _CHUNK, 8), jnp.float32),
            pltpu.VMEM((_CPW, _CHUNK, 1), jnp.float32),
            pltpu.VMEM((_CPW, _CHUNK, 1), jnp.float32),
            pltpu.SemaphoreType.DMA,
        ],
    )
    def k(uidx_hbm, iidx_hbm, tidx_hbm, due_hbm, die_hbm, dte_hbm, wue_hbm,
          wie_hbm, u_out, i_out, t_out, wu_out, wi_out,
          uidx_v, iidx_v, tidx_v, u_v, i_v, t_v, wu_v, wi_v, sem):
        wid = lax.axis_index("s") * _NC + lax.axis_index("c")
        base = wid * _CPW
        pltpu.sync_copy(uidx_hbm.at[pl.ds(base, _CPW)], uidx_v)
        pltpu.sync_copy(iidx_hbm.at[pl.ds(base, _CPW)], iidx_v)
        pltpu.sync_copy(tidx_hbm.at[pl.ds(base, _CPW)], tidx_v)
        copies = []
        for j in range(_CPW):
            copies.append(pltpu.async_copy(due_hbm.at[uidx_v.at[j]], u_v.at[j], sem))
            copies.append(pltpu.async_copy(die_hbm.at[iidx_v.at[j]], i_v.at[j], sem))
            copies.append(pltpu.async_copy(dte_hbm.at[tidx_v.at[j]], t_v.at[j], sem))
            copies.append(pltpu.async_copy(wue_hbm.at[uidx_v.at[j]], wu_v.at[j], sem))
            copies.append(pltpu.async_copy(wie_hbm.at[iidx_v.at[j]], wi_v.at[j], sem))
        for c in copies:
            c.wait()
        pltpu.sync_copy(u_v, u_out.at[pl.ds(base, _CPW)])
        pltpu.sync_copy(i_v, i_out.at[pl.ds(base, _CPW)])
        pltpu.sync_copy(t_v, t_out.at[pl.ds(base, _CPW)])
        pltpu.sync_copy(wu_v, wu_out.at[pl.ds(base, _CPW)])
        pltpu.sync_copy(wi_v, wi_out.at[pl.ds(base, _CPW)])

    return k(uidx2, iidx2, tidx2, due, die, dte, wue, wie)


def _bn(h, g, be):
    m = jnp.mean(h, axis=0, keepdims=True)
    v = jnp.mean((h - m) ** 2, axis=0, keepdims=True)
    return (h - m) * lax.rsqrt(v + 1e-5) * g + be


def _mlp_body(x, wb, w1, b1, g1, be1,
              w2, b2, g2, be2, w3, b3, g3, be3, w4, b4, out):
    f32 = jnp.float32
    xv = x[...]
    h = jnp.dot(xv, w1[...], preferred_element_type=f32) + b1[...]
    h = _bn(jnp.maximum(h, 0.0), g1[...], be1[...])
    h = jnp.maximum(jnp.dot(h, w2[...], preferred_element_type=f32) + b2[...], 0.0)
    h = _bn(h, g2[...], be2[...])
    h = jnp.maximum(jnp.dot(h, w3[...], preferred_element_type=f32) + b3[...], 0.0)
    h = _bn(h, g3[...], be3[...])
    deep = jnp.dot(h, w4[...], preferred_element_type=f32) + b4[...]
    z = deep + xv[:, 73:74] + xv[:, 74:75] + wb[...]
    out[...] = jax.nn.sigmoid(z)


def kernel(user_idx, item_idx, type_idx, price_norm, wide_user_emb,
           wide_item_emb, wide_bias, deep_user_emb, deep_item_emb,
           deep_type_emb, W1, b1, g1, be1, W2, b2, g2, be2, W3, b3, g3, be3,
           W4, b4):
    uidx2 = user_idx.astype(jnp.int32).reshape(_NCHUNK, _CHUNK)
    iidx2 = item_idx.astype(jnp.int32).reshape(_NCHUNK, _CHUNK)
    tidx2 = type_idx.astype(jnp.int32).reshape(_NCHUNK, _CHUNK)
    u, i, t, wu, wi = _sc_gather(uidx2, iidx2, tidx2, deep_user_emb,
                                 deep_item_emb, deep_type_emb,
                                 wide_user_emb, wide_item_emb)
    t = jnp.take(deep_type_emb, type_idx, axis=0)
    wu = jnp.take(wide_user_emb, user_idx, axis=0)
    wi = jnp.take(wide_item_emb, item_idx, axis=0)
    x = jnp.concatenate([
        u.reshape(_B, 32), i.reshape(_B, 32), t.reshape(_B, 8),
        price_norm.reshape(_B, 1), wu.reshape(_B, 1), wi.reshape(_B, 1),
    ], axis=1)
    w1p = jnp.concatenate([W1, jnp.zeros((2, 128), jnp.float32)], axis=0)
    out = pl.pallas_call(
        _mlp_body,
        out_shape=jax.ShapeDtypeStruct((_B, 1), jnp.float32),
    )(x, wide_bias.reshape(1, 1), w1p,
      b1.reshape(1, -1), g1.reshape(1, -1), be1.reshape(1, -1),
      W2, b2.reshape(1, -1), g2.reshape(1, -1), be2.reshape(1, -1),
      W3, b3.reshape(1, -1), g3.reshape(1, -1), be3.reshape(1, -1),
      W4, b4.reshape(1, 1))
    return out[:, 0]


# speedup vs baseline: 2.4970x; 2.4970x over previous
import functools

import jax
import jax.numpy as jnp
from jax import lax
from jax.experimental import pallas as pl
from jax.experimental.pallas import tpu as pltpu
from jax.experimental.pallas import tpu_sc as plsc

_NC, _NS = 2, 16
_NW = _NC * _NS
_B = 16384
_RPW = _B // _NW
_IC = _RPW // 128
_SUB = 256


def _sc_gather(uidx3, iidx3, tidx3, due, die, dte, wue, wie):
    mesh = plsc.VectorSubcoreMesh(core_axis_name="c", subcore_axis_name="s")

    @functools.partial(
        pl.kernel,
        out_type=[
            jax.ShapeDtypeStruct((_B, 32), jnp.float32),
            jax.ShapeDtypeStruct((_B, 32), jnp.float32),
            jax.ShapeDtypeStruct((_B, 8), jnp.float32),
            jax.ShapeDtypeStruct((_B, 1), jnp.float32),
            jax.ShapeDtypeStruct((_B, 1), jnp.float32),
        ],
        mesh=mesh,
        scratch_types=[
            pltpu.VMEM((_IC, 128), jnp.int32),
            pltpu.VMEM((_IC, 128), jnp.int32),
            pltpu.VMEM((_IC, 128), jnp.int32),
            pltpu.VMEM((_SUB, 32), jnp.float32),
            pltpu.VMEM((_SUB, 8), jnp.float32),
            pltpu.VMEM((_SUB, 1), jnp.float32),
            pltpu.SemaphoreType.DMA,
        ],
    )
    def k(uidx_hbm, iidx_hbm, tidx_hbm, due_hbm, die_hbm, dte_hbm, wue_hbm,
          wie_hbm, u_out, i_out, t_out, wu_out, wi_out,
          uidx_sm, iidx_sm, tidx_sm, gb32, gb8, gb1, sem):
        wid = lax.axis_index("s") * _NC + lax.axis_index("c")
        base = wid * _RPW

        pltpu.sync_copy(uidx_hbm.at[wid], uidx_sm)
        pltpu.sync_copy(iidx_hbm.at[wid], iidx_sm)
        pltpu.sync_copy(tidx_hbm.at[wid], tidx_sm)

        def gather_phase(idx_sm, table, gb, out):
            for s in range(_RPW // _SUB):
                for j in range(s * (_SUB // 128), (s + 1) * (_SUB // 128)):
                    def body(g, _, j=j, s=s):
                        vec = idx_sm[j, pl.ds(pl.multiple_of(g * 16, 16), 16)]
                        for l in range(16):
                            row = vec[l]
                            pos = (j - s * (_SUB // 128)) * 128 + g * 16 + l
                            pltpu.async_copy(
                                table.at[pl.ds(row, 1), :],
                                gb.at[pl.ds(pos, 1), :],
                                sem,
                            )
                        return _
                    lax.fori_loop(0, 8, body, None)
                pltpu.make_async_copy(
                    table.at[pl.ds(0, _SUB), :], gb, sem
                ).wait()
                pltpu.sync_copy(gb, out.at[pl.ds(base + s * _SUB, _SUB), :])

        gather_phase(uidx_sm, due_hbm, gb32, u_out)
        gather_phase(iidx_sm, die_hbm, gb32, i_out)
        gather_phase(tidx_sm, dte_hbm, gb8, t_out)
        gather_phase(uidx_sm, wue_hbm, gb1, wu_out)
        gather_phase(iidx_sm, wie_hbm, gb1, wi_out)

    return k(uidx3, iidx3, tidx3, due, die, dte, wue, wie)


def _bn(h, g, be):
    m = jnp.mean(h, axis=0, keepdims=True)
    v = jnp.mean((h - m) ** 2, axis=0, keepdims=True)
    return (h - m) * lax.rsqrt(v + 1e-5) * g + be


def _mlp_body(x, wb, w1, b1, g1, be1,
              w2, b2, g2, be2, w3, b3, g3, be3, w4, b4, out):
    f32 = jnp.float32
    xv = x[...]
    h = jnp.dot(xv, w1[...], preferred_element_type=f32) + b1[...]
    h = _bn(jnp.maximum(h, 0.0), g1[...], be1[...])
    h = jnp.maximum(jnp.dot(h, w2[...], preferred_element_type=f32) + b2[...], 0.0)
    h = _bn(h, g2[...], be2[...])
    h = jnp.maximum(jnp.dot(h, w3[...], preferred_element_type=f32) + b3[...], 0.0)
    h = _bn(h, g3[...], be3[...])
    deep = jnp.dot(h, w4[...], preferred_element_type=f32) + b4[...]
    z = deep + xv[:, 73:74] + xv[:, 74:75] + wb[...]
    out[...] = jax.nn.sigmoid(z)


def kernel(user_idx, item_idx, type_idx, price_norm, wide_user_emb,
           wide_item_emb, wide_bias, deep_user_emb, deep_item_emb,
           deep_type_emb, W1, b1, g1, be1, W2, b2, g2, be2, W3, b3, g3, be3,
           W4, b4):
    uidx3 = user_idx.astype(jnp.int32).reshape(_NW, _IC, 128)
    iidx3 = item_idx.astype(jnp.int32).reshape(_NW, _IC, 128)
    tidx3 = type_idx.astype(jnp.int32).reshape(_NW, _IC, 128)
    u, i, t, wu, wi = _sc_gather(uidx3, iidx3, tidx3, deep_user_emb,
                                 deep_item_emb, deep_type_emb,
                                 wide_user_emb, wide_item_emb)
    x = jnp.concatenate(
        [u, i, t, price_norm.reshape(_B, 1), wu, wi], axis=1)
    w1p = jnp.concatenate([W1, jnp.zeros((2, 128), jnp.float32)], axis=0)
    out = pl.pallas_call(
        _mlp_body,
        out_shape=jax.ShapeDtypeStruct((_B, 1), jnp.float32),
    )(x, wide_bias.reshape(1, 1), w1p,
      b1.reshape(1, -1), g1.reshape(1, -1), be1.reshape(1, -1),
      W2, b2.reshape(1, -1), g2.reshape(1, -1), be2.reshape(1, -1),
      W3, b3.reshape(1, -1), g3.reshape(1, -1), be3.reshape(1, -1),
      W4, b4.reshape(1, 1))
    return out[:, 0]


# speedup vs baseline: 2.4986x; 1.0006x over previous
import functools

import jax
import jax.numpy as jnp
from jax import lax
from jax.experimental import pallas as pl
from jax.experimental.pallas import tpu as pltpu
from jax.experimental.pallas import tpu_sc as plsc

_NC, _NS = 2, 16
_NW = _NC * _NS
_B = 16384
_RPW = _B // _NW
_SUB = 256


def _sc_gather(uidx, iidx, tidx, due, die, dte, wue, wie):
    mesh = plsc.VectorSubcoreMesh(core_axis_name="c", subcore_axis_name="s")

    @functools.partial(
        pl.kernel,
        out_type=[
            jax.ShapeDtypeStruct((_B, 32), jnp.float32),
            jax.ShapeDtypeStruct((_B, 32), jnp.float32),
            jax.ShapeDtypeStruct((_B, 8), jnp.float32),
            jax.ShapeDtypeStruct((_B, 1), jnp.float32),
            jax.ShapeDtypeStruct((_B, 1), jnp.float32),
        ],
        mesh=mesh,
        scratch_types=[
            pltpu.VMEM((_RPW,), jnp.int32),
            pltpu.VMEM((_RPW,), jnp.int32),
            pltpu.VMEM((_RPW,), jnp.int32),
            pltpu.VMEM((_SUB, 32), jnp.float32),
            pltpu.VMEM((_SUB, 8), jnp.float32),
            pltpu.VMEM((_SUB, 1), jnp.float32),
            pltpu.SemaphoreType.DMA,
        ],
    )
    def k(uidx_hbm, iidx_hbm, tidx_hbm, due_hbm, die_hbm, dte_hbm, wue_hbm,
          wie_hbm, u_out, i_out, t_out, wu_out, wi_out,
          uidx_sm, iidx_sm, tidx_sm, gb32, gb8, gb1, sem):
        wid = lax.axis_index("s") * _NC + lax.axis_index("c")
        base = wid * _RPW

        pltpu.sync_copy(uidx_hbm.at[pl.ds(base, _RPW)], uidx_sm)
        pltpu.sync_copy(iidx_hbm.at[pl.ds(base, _RPW)], iidx_sm)
        pltpu.sync_copy(tidx_hbm.at[pl.ds(base, _RPW)], tidx_sm)

        def gather_phase(idx_sm, table, gb, out):
            for s in range(_RPW // _SUB):
                def body(g, _, s=s):
                    off = pl.multiple_of(s * _SUB + g * 16, 16)
                    vec = idx_sm[pl.ds(off, 16)]
                    for l in range(16):
                        pos = g * 16 + l
                        pltpu.async_copy(
                            table.at[pl.ds(vec[l], 1), :],
                            gb.at[pl.ds(pos, 1), :],
                            sem,
                        )
                    return _
                lax.fori_loop(0, _SUB // 16, body, None)
                pltpu.make_async_copy(
                    table.at[pl.ds(0, _SUB), :], gb, sem
                ).wait()
                pltpu.sync_copy(gb, out.at[pl.ds(base + s * _SUB, _SUB), :])

        gather_phase(uidx_sm, due_hbm, gb32, u_out)
        gather_phase(iidx_sm, die_hbm, gb32, i_out)
        gather_phase(tidx_sm, dte_hbm, gb8, t_out)
        gather_phase(uidx_sm, wue_hbm, gb1, wu_out)
        gather_phase(iidx_sm, wie_hbm, gb1, wi_out)

    return k(uidx, iidx, tidx, due, die, dte, wue, wie)


def _bn(h, g, be):
    m = jnp.mean(h, axis=0, keepdims=True)
    v = jnp.mean((h - m) ** 2, axis=0, keepdims=True)
    return (h - m) * lax.rsqrt(v + 1e-5) * g + be


def _mlp_body(x, wb, w1, b1, g1, be1,
              w2, b2, g2, be2, w3, b3, g3, be3, w4, b4, out):
    f32 = jnp.float32
    xv = x[...]
    h = jnp.dot(xv, w1[...], preferred_element_type=f32) + b1[...]
    h = _bn(jnp.maximum(h, 0.0), g1[...], be1[...])
    h = jnp.maximum(jnp.dot(h, w2[...], preferred_element_type=f32) + b2[...], 0.0)
    h = _bn(h, g2[...], be2[...])
    h = jnp.maximum(jnp.dot(h, w3[...], preferred_element_type=f32) + b3[...], 0.0)
    h = _bn(h, g3[...], be3[...])
    deep = jnp.dot(h, w4[...], preferred_element_type=f32) + b4[...]
    z = deep + xv[:, 73:74] + xv[:, 74:75] + wb[...]
    out[...] = jax.nn.sigmoid(z)


def kernel(user_idx, item_idx, type_idx, price_norm, wide_user_emb,
           wide_item_emb, wide_bias, deep_user_emb, deep_item_emb,
           deep_type_emb, W1, b1, g1, be1, W2, b2, g2, be2, W3, b3, g3, be3,
           W4, b4):
    u, i, t, wu, wi = _sc_gather(user_idx.astype(jnp.int32),
                                 item_idx.astype(jnp.int32),
                                 type_idx.astype(jnp.int32),
                                 deep_user_emb, deep_item_emb, deep_type_emb,
                                 wide_user_emb, wide_item_emb)
    x = jnp.concatenate(
        [u, i, t, price_norm.reshape(_B, 1), wu, wi], axis=1)
    w1p = jnp.concatenate([W1, jnp.zeros((2, 128), jnp.float32)], axis=0)
    out = pl.pallas_call(
        _mlp_body,
        out_shape=jax.ShapeDtypeStruct((_B, 1), jnp.float32),
    )(x, wide_bias.reshape(1, 1), w1p,
      b1.reshape(1, -1), g1.reshape(1, -1), be1.reshape(1, -1),
      W2, b2.reshape(1, -1), g2.reshape(1, -1), be2.reshape(1, -1),
      W3, b3.reshape(1, -1), g3.reshape(1, -1), be3.reshape(1, -1),
      W4, b4.reshape(1, 1))
    return out[:, 0]
